# baseline (speedup 1.0000x reference)
import functools

import jax
import jax.numpy as jnp
from jax import lax
from jax.experimental import pallas as pl
from jax.experimental.pallas import tpu as pltpu
from jax.experimental.pallas import tpu_sc as plsc

V = 1000
C = 1000
CP = 1024
NC, NS = 2, 16
NW = NC * NS
CH = 16
NBUF = 4
BB = 1024


def _lse_body(table_ref, out_ref):
    x = table_ref[...]
    m = jnp.max(x, axis=1, keepdims=True)
    s = jnp.sum(jnp.exp(x - m), axis=1, keepdims=True)
    out_ref[...] = m + jnp.log(s)


def _row_lse(table):
    return pl.pallas_call(
        _lse_body,
        out_shape=jax.ShapeDtypeStruct((V, 1), jnp.float32),
    )(table)


def _loss_body(*refs, inv_n):
    out_ref = refs[-1]
    s = sum(jnp.sum(r[...]) for r in refs[:-1]) * inv_n
    out_ref[...] = jnp.broadcast_to(s, (1, 1))


def _finalize_loss(partials_list, n):
    out = pl.pallas_call(
        functools.partial(_loss_body, inv_n=1.0 / n),
        out_shape=jax.ShapeDtypeStruct((1, 1), jnp.float32),
    )(*partials_list)
    return out[0, 0]


def _tr_compute(in_ref, out_ref):
    x = in_ref[...]
    out_ref[0] = x.reshape(BB, CP)[:, :C].T


def _tr_body2(in_ref, out_ref):
    _tr_compute(in_ref, out_ref)


def _tr_body_alias(in_ref, prev_ref, out_ref):
    del prev_ref
    _tr_compute(in_ref, out_ref)


def _tc_transpose_part(out1h, prev, ta, tspan, B, T):
    nb = B // BB
    out_spec = pl.BlockSpec((1, C, BB), lambda t, b: (t + ta, 0, b))
    in_spec = pl.BlockSpec((BB * 8, 128), lambda t, b: (t * nb + b, 0))
    out_shape = jax.ShapeDtypeStruct((T, C, B), jnp.float32)
    if prev is None:
        return pl.pallas_call(
            _tr_body2,
            grid=(tspan, nb),
            in_specs=[in_spec],
            out_specs=out_spec,
            out_shape=out_shape,
        )(out1h)
    return pl.pallas_call(
        _tr_body_alias,
        grid=(tspan, nb),
        in_specs=[in_spec, pl.BlockSpec(memory_space=pl.ANY)],
        out_specs=out_spec,
        out_shape=out_shape,
        input_output_aliases={1: 0},
    )(out1h, prev)


def _sc_gather(tab_p, idxT, tgtT, lse, t_base, tspan):
    T, B = idxT.shape
    n_chunks_b = B // CH
    per_w = (tspan * n_chunks_b) // NW
    mesh = plsc.VectorSubcoreMesh(
        core_axis_name="c", subcore_axis_name="s",
        num_cores=NC, num_subcores=NS)

    @functools.partial(
        pl.kernel,
        out_type=[jax.ShapeDtypeStruct((tspan * B, CP), jnp.float32),
                  jax.ShapeDtypeStruct((NW, 16), jnp.float32)],
        mesh=mesh,
        compiler_params=pltpu.CompilerParams(
            needs_layout_passes=False, use_tc_tiling_on_sc=False),
        scratch_types=[
            pltpu.VMEM_SHARED((V, CP), jnp.float32),
            pltpu.VMEM((NBUF, CH), jnp.int32),
            pltpu.VMEM((NBUF, CH), jnp.int32),
            pltpu.VMEM((V,), jnp.float32),
            pltpu.VMEM((NBUF, CH, CP), jnp.float32),
            pltpu.VMEM((16,), jnp.float32),
            pltpu.SemaphoreType.DMA,
            pltpu.SemaphoreType.DMA,
            pltpu.SemaphoreType.DMA,
            pltpu.SemaphoreType.DMA,
            pltpu.SemaphoreType.DMA,
            pltpu.SemaphoreType.DMA,
            pltpu.SemaphoreType.DMA,
            pltpu.SemaphoreType.DMA,
        ],
    )
    def k(tab_hbm, idx_hbm, tgt_hbm, lse_hbm, out_hbm, part_hbm,
          table_sh, idx_v, tgt_v, lse_v, rows_v, acc_v,
          g0, g1, g2, g3, o0, o1, o2, o3):
        cid = lax.axis_index("c")
        sid = lax.axis_index("s")
        wid = sid * NC + cid
        gsem = (g0, g1, g2, g3)
        osem = (o0, o1, o2, o3)

        @pl.when(sid == 0)
        def _():
            pltpu.sync_copy(tab_hbm, table_sh)
        plsc.subcore_barrier()

        pltpu.sync_copy(lse_hbm, lse_v)
        acc_v[...] = jnp.zeros((16,), jnp.float32)
        lanes = lax.iota(jnp.int32, 16)
        lo = wid * per_w
        hi = lo + per_w

        def start_gather(u, s):
            t = t_base + u // n_chunks_b
            b0 = (u % n_chunks_b) * CH
            pltpu.sync_copy(idx_hbm.at[t, pl.ds(b0, CH)], idx_v.at[s])
            pltpu.sync_copy(tgt_hbm.at[t, pl.ds(b0, CH)], tgt_v.at[s])
            pltpu.async_copy(table_sh.at[idx_v.at[s]], rows_v.at[s], gsem[s])

        for s in range(NBUF):
            start_gather(lo + s, s)

        def pair(i, carry):
            u0 = lo + i * NBUF
            for s in range(NBUF):
                u = u0 + s
                t = u // n_chunks_b
                b0 = (u % n_chunks_b) * CH
                pltpu.make_async_copy(
                    table_sh.at[idx_v.at[s]], rows_v.at[s], gsem[s]).wait()
                acc = acc_v[...]
                for g in range(CH // 16):
                    ii = idx_v[s, pl.ds(g * 16, 16)]
                    tt = tgt_v[s, pl.ds(g * 16, 16)]
                    lse_g = plsc.load_gather(lse_v, [ii])
                    r = lanes + (g * 16)
                    tv = plsc.load_gather(rows_v.at[s], [r, tt])
                    acc = acc + (lse_g - tv)
                acc_v[...] = acc
                pltpu.async_copy(
                    rows_v.at[s], out_hbm.at[pl.ds(t * B + b0, CH)], osem[s])
            for s in range(NBUF):
                u = u0 + s
                un = u + NBUF
                t = u // n_chunks_b
                b0 = (u % n_chunks_b) * CH
                dst = out_hbm.at[pl.ds(t * B + b0, CH)]

                @pl.when(un < hi)
                def _():
                    pltpu.make_async_copy(rows_v.at[s], dst, osem[s]).wait()
                    start_gather(un, s)
            return carry

        lax.fori_loop(0, per_w // NBUF, pair, 0, unroll=False)

        for s in range(NBUF):
            u = hi - NBUF + s
            t = u // n_chunks_b
            b0 = (u % n_chunks_b) * CH
            dst = out_hbm.at[pl.ds(t * B + b0, CH)]
            pltpu.make_async_copy(rows_v.at[s], dst, osem[s]).wait()
        pltpu.sync_copy(acc_v, part_hbm.at[wid])

    return k(tab_p, idxT, tgtT, lse)


SPLITS = ((0, 25), (25, 25))


def kernel(idx, targets, table):
    Bq, Tq = idx.shape
    n = Bq * Tq
    idxT = idx.astype(jnp.int32).T
    tgtT = targets.astype(jnp.int32).T
    table = table.astype(jnp.float32)
    tab_p = jnp.pad(table, ((0, 0), (0, CP - C)))
    lse = _row_lse(table).reshape(V)
    halves = [_sc_gather(tab_p, idxT, tgtT, lse, ta, ts)
              for ta, ts in SPLITS]
    out3 = None
    for (ta, ts), (o, _) in zip(SPLITS, halves):
        o8 = o.reshape(ts * Bq * 8, 128)
        out3 = _tc_transpose_part(o8, out3, ta, ts, Bq, Tq)
    logits = jnp.transpose(out3, (2, 0, 1))
    loss = _finalize_loss([p for _, p in halves], n)
    return logits, loss


# speedup vs baseline: 1.1032x; 1.1032x over previous
import functools

import jax
import jax.numpy as jnp
from jax import lax
from jax.experimental import pallas as pl
from jax.experimental.pallas import tpu as pltpu
from jax.experimental.pallas import tpu_sc as plsc

V = 1000
C = 1000
CP = 1024
NC, NS = 2, 16
NW = NC * NS
CH = 32
NBUF = 2
BB = 1024


def _lse_body(table_ref, out_ref):
    x = table_ref[...]
    m = jnp.max(x, axis=1, keepdims=True)
    s = jnp.sum(jnp.exp(x - m), axis=1, keepdims=True)
    out_ref[...] = m + jnp.log(s)


def _row_lse(table):
    return pl.pallas_call(
        _lse_body,
        out_shape=jax.ShapeDtypeStruct((V, 1), jnp.float32),
    )(table)


def _loss_body(*refs, inv_n):
    out_ref = refs[-1]
    s = sum(jnp.sum(r[...]) for r in refs[:-1]) * inv_n
    out_ref[...] = jnp.broadcast_to(s, (1, 1))


def _finalize_loss(partials_list, n):
    out = pl.pallas_call(
        functools.partial(_loss_body, inv_n=1.0 / n),
        out_shape=jax.ShapeDtypeStruct((1, 1), jnp.float32),
    )(*partials_list)
    return out[0, 0]


def _tr_compute(in_ref, out_ref):
    x = in_ref[...]
    out_ref[0] = x.reshape(BB, CP)[:, :C].T


def _tr_body2(in_ref, out_ref):
    _tr_compute(in_ref, out_ref)


def _tr_body_alias(in_ref, prev_ref, out_ref):
    del prev_ref
    _tr_compute(in_ref, out_ref)


def _tc_transpose_part(out1h, prev, ta, tspan, B, T):
    nb = B // BB
    out_spec = pl.BlockSpec((1, C, BB), lambda t, b: (t + ta, 0, b))
    in_spec = pl.BlockSpec((BB * 8, 128), lambda t, b: (t * nb + b, 0))
    out_shape = jax.ShapeDtypeStruct((T, C, B), jnp.float32)
    if prev is None:
        return pl.pallas_call(
            _tr_body2,
            grid=(tspan, nb),
            in_specs=[in_spec],
            out_specs=out_spec,
            out_shape=out_shape,
        )(out1h)
    return pl.pallas_call(
        _tr_body_alias,
        grid=(tspan, nb),
        in_specs=[in_spec, pl.BlockSpec(memory_space=pl.ANY)],
        out_specs=out_spec,
        out_shape=out_shape,
        input_output_aliases={1: 0},
    )(out1h, prev)


def _sc_gather(tab_p, idxT, tgtT, lse, t_base, tspan):
    T, B = idxT.shape
    n_chunks_b = B // CH
    per_w = (tspan * n_chunks_b) // NW
    mesh = plsc.VectorSubcoreMesh(
        core_axis_name="c", subcore_axis_name="s",
        num_cores=NC, num_subcores=NS)

    @functools.partial(
        pl.kernel,
        out_type=[jax.ShapeDtypeStruct((tspan * B, CP), jnp.float32),
                  jax.ShapeDtypeStruct((NW, 16), jnp.float32)],
        mesh=mesh,
        compiler_params=pltpu.CompilerParams(
            needs_layout_passes=False, use_tc_tiling_on_sc=False),
        scratch_types=[
            pltpu.VMEM_SHARED((V, CP), jnp.float32),
            pltpu.VMEM((NBUF, CH), jnp.int32),
            pltpu.VMEM((NBUF, CH), jnp.int32),
            pltpu.VMEM((V,), jnp.float32),
            pltpu.VMEM((NBUF, CH, CP), jnp.float32),
            pltpu.VMEM((16,), jnp.float32),
            pltpu.SemaphoreType.DMA,
            pltpu.SemaphoreType.DMA,
            pltpu.SemaphoreType.DMA,
            pltpu.SemaphoreType.DMA,
        ],
    )
    def k(tab_hbm, idx_hbm, tgt_hbm, lse_hbm, out_hbm, part_hbm,
          table_sh, idx_v, tgt_v, lse_v, rows_v, acc_v, g0, g1, o0, o1):
        cid = lax.axis_index("c")
        sid = lax.axis_index("s")
        wid = sid * NC + cid
        gsem = (g0, g1)
        osem = (o0, o1)

        @pl.when(sid == 0)
        def _():
            pltpu.sync_copy(tab_hbm, table_sh)
        plsc.subcore_barrier()

        pltpu.sync_copy(lse_hbm, lse_v)
        acc_v[...] = jnp.zeros((16,), jnp.float32)
        lanes = lax.iota(jnp.int32, 16)
        lo = wid * per_w
        hi = lo + per_w

        def start_gather(u, s):
            t = t_base + u // n_chunks_b
            b0 = (u % n_chunks_b) * CH
            pltpu.sync_copy(idx_hbm.at[t, pl.ds(b0, CH)], idx_v.at[s])
            pltpu.sync_copy(tgt_hbm.at[t, pl.ds(b0, CH)], tgt_v.at[s])
            pltpu.async_copy(table_sh.at[idx_v.at[s]], rows_v.at[s], gsem[s])

        for s in range(NBUF):
            start_gather(lo + s, s)

        def pair(i, carry):
            u0 = lo + i * NBUF
            for s in range(NBUF):
                u = u0 + s
                t = u // n_chunks_b
                b0 = (u % n_chunks_b) * CH
                pltpu.make_async_copy(
                    table_sh.at[idx_v.at[s]], rows_v.at[s], gsem[s]).wait()
                acc = acc_v[...]
                for g in range(CH // 16):
                    ii = idx_v[s, pl.ds(g * 16, 16)]
                    tt = tgt_v[s, pl.ds(g * 16, 16)]
                    lse_g = plsc.load_gather(lse_v, [ii])
                    r = lanes + (g * 16)
                    tv = plsc.load_gather(rows_v.at[s], [r, tt])
                    acc = acc + (lse_g - tv)
                acc_v[...] = acc
                pltpu.async_copy(
                    rows_v.at[s], out_hbm.at[pl.ds(t * B + b0, CH)], osem[s])
            for s in range(NBUF):
                u = u0 + s
                un = u + NBUF
                t = u // n_chunks_b
                b0 = (u % n_chunks_b) * CH
                dst = out_hbm.at[pl.ds(t * B + b0, CH)]

                @pl.when(un < hi)
                def _():
                    pltpu.make_async_copy(rows_v.at[s], dst, osem[s]).wait()
                    start_gather(un, s)
            return carry

        lax.fori_loop(0, per_w // NBUF, pair, 0, unroll=False)

        for s in range(NBUF):
            u = hi - NBUF + s
            t = u // n_chunks_b
            b0 = (u % n_chunks_b) * CH
            dst = out_hbm.at[pl.ds(t * B + b0, CH)]
            pltpu.make_async_copy(rows_v.at[s], dst, osem[s]).wait()
        pltpu.sync_copy(acc_v, part_hbm.at[wid])

    return k(tab_p, idxT, tgtT, lse)


SPLITS = ((0, 25), (25, 25))


def kernel(idx, targets, table):
    Bq, Tq = idx.shape
    n = Bq * Tq
    idxT = idx.astype(jnp.int32).T
    tgtT = targets.astype(jnp.int32).T
    table = table.astype(jnp.float32)
    tab_p = jnp.pad(table, ((0, 0), (0, CP - C)))
    lse = _row_lse(table).reshape(V)
    halves = [_sc_gather(tab_p, idxT, tgtT, lse, ta, ts)
              for ta, ts in SPLITS]
    out3 = None
    for (ta, ts), (o, _) in zip(SPLITS, halves):
        o8 = o.reshape(ts * Bq * 8, 128)
        out3 = _tc_transpose_part(o8, out3, ta, ts, Bq, Tq)
    logits = jnp.transpose(out3, (2, 0, 1))
    loss = _finalize_loss([p for _, p in halves], n)
    return logits, loss


# speedup vs baseline: 1.1233x; 1.0183x over previous
import functools

import jax
import jax.numpy as jnp
from jax import lax
from jax.experimental import pallas as pl
from jax.experimental.pallas import tpu as pltpu
from jax.experimental.pallas import tpu_sc as plsc

V = 1000
C = 1000
CP = 1024
NC, NS = 2, 16
NW = NC * NS
CH = 32
NBUF = 2
BB = 2048


def _lse_body(table_ref, out_ref):
    x = table_ref[...]
    m = jnp.max(x, axis=1, keepdims=True)
    s = jnp.sum(jnp.exp(x - m), axis=1, keepdims=True)
    out_ref[...] = m + jnp.log(s)


def _row_lse(table):
    return pl.pallas_call(
        _lse_body,
        out_shape=jax.ShapeDtypeStruct((V, 1), jnp.float32),
    )(table)


def _loss_body(*refs, inv_n):
    out_ref = refs[-1]
    s = sum(jnp.sum(r[...]) for r in refs[:-1]) * inv_n
    out_ref[...] = jnp.broadcast_to(s, (1, 1))


def _finalize_loss(partials_list, n):
    out = pl.pallas_call(
        functools.partial(_loss_body, inv_n=1.0 / n),
        out_shape=jax.ShapeDtypeStruct((1, 1), jnp.float32),
    )(*partials_list)
    return out[0, 0]


def _tr_compute(in_ref, out_ref):
    x = in_ref[...]
    out_ref[0] = x.reshape(BB, CP)[:, :C].T


def _tr_body2(in_ref, out_ref):
    _tr_compute(in_ref, out_ref)


def _tr_body_alias(in_ref, prev_ref, out_ref):
    del prev_ref
    _tr_compute(in_ref, out_ref)


def _tc_transpose_part(out1h, prev, ta, tspan, B, T):
    nb = B // BB
    out_spec = pl.BlockSpec((1, C, BB), lambda t, b: (t + ta, 0, b))
    in_spec = pl.BlockSpec((BB * 8, 128), lambda t, b: (t * nb + b, 0))
    out_shape = jax.ShapeDtypeStruct((T, C, B), jnp.float32)
    if prev is None:
        return pl.pallas_call(
            _tr_body2,
            grid=(tspan, nb),
            in_specs=[in_spec],
            out_specs=out_spec,
            out_shape=out_shape,
        )(out1h)
    return pl.pallas_call(
        _tr_body_alias,
        grid=(tspan, nb),
        in_specs=[in_spec, pl.BlockSpec(memory_space=pl.ANY)],
        out_specs=out_spec,
        out_shape=out_shape,
        input_output_aliases={1: 0},
    )(out1h, prev)


def _sc_gather(tab_p, idxT, tgtT, lse, t_base, tspan):
    T, B = idxT.shape
    n_chunks_b = B // CH
    per_w = (tspan * n_chunks_b) // NW
    mesh = plsc.VectorSubcoreMesh(
        core_axis_name="c", subcore_axis_name="s",
        num_cores=NC, num_subcores=NS)

    @functools.partial(
        pl.kernel,
        out_type=[jax.ShapeDtypeStruct((tspan * B, CP), jnp.float32),
                  jax.ShapeDtypeStruct((NW, 16), jnp.float32)],
        mesh=mesh,
        compiler_params=pltpu.CompilerParams(
            needs_layout_passes=False, use_tc_tiling_on_sc=False),
        scratch_types=[
            pltpu.VMEM_SHARED((V, CP), jnp.float32),
            pltpu.VMEM((NBUF, CH), jnp.int32),
            pltpu.VMEM((NBUF, CH), jnp.int32),
            pltpu.VMEM((V,), jnp.float32),
            pltpu.VMEM((NBUF, CH, CP), jnp.float32),
            pltpu.VMEM((16,), jnp.float32),
            pltpu.SemaphoreType.DMA,
            pltpu.SemaphoreType.DMA,
            pltpu.SemaphoreType.DMA,
            pltpu.SemaphoreType.DMA,
        ],
    )
    def k(tab_hbm, idx_hbm, tgt_hbm, lse_hbm, out_hbm, part_hbm,
          table_sh, idx_v, tgt_v, lse_v, rows_v, acc_v, g0, g1, o0, o1):
        cid = lax.axis_index("c")
        sid = lax.axis_index("s")
        wid = sid * NC + cid
        gsem = (g0, g1)
        osem = (o0, o1)

        @pl.when(sid == 0)
        def _():
            pltpu.sync_copy(tab_hbm, table_sh)
        plsc.subcore_barrier()

        pltpu.sync_copy(lse_hbm, lse_v)
        acc_v[...] = jnp.zeros((16,), jnp.float32)
        lanes = lax.iota(jnp.int32, 16)
        lo = wid * per_w
        hi = lo + per_w

        def start_gather(u, s):
            t = t_base + u // n_chunks_b
            b0 = (u % n_chunks_b) * CH
            pltpu.sync_copy(idx_hbm.at[t, pl.ds(b0, CH)], idx_v.at[s])
            pltpu.sync_copy(tgt_hbm.at[t, pl.ds(b0, CH)], tgt_v.at[s])
            pltpu.async_copy(table_sh.at[idx_v.at[s]], rows_v.at[s], gsem[s])

        for s in range(NBUF):
            start_gather(lo + s, s)

        def pair(i, carry):
            u0 = lo + i * NBUF
            for s in range(NBUF):
                u = u0 + s
                t = u // n_chunks_b
                b0 = (u % n_chunks_b) * CH
                pltpu.make_async_copy(
                    table_sh.at[idx_v.at[s]], rows_v.at[s], gsem[s]).wait()
                acc = acc_v[...]
                for g in range(CH // 16):
                    ii = idx_v[s, pl.ds(g * 16, 16)]
                    tt = tgt_v[s, pl.ds(g * 16, 16)]
                    lse_g = plsc.load_gather(lse_v, [ii])
                    r = lanes + (g * 16)
                    tv = plsc.load_gather(rows_v.at[s], [r, tt])
                    acc = acc + (lse_g - tv)
                acc_v[...] = acc
                pltpu.async_copy(
                    rows_v.at[s], out_hbm.at[pl.ds(t * B + b0, CH)], osem[s])
            for s in range(NBUF):
                u = u0 + s
                un = u + NBUF
                t = u // n_chunks_b
                b0 = (u % n_chunks_b) * CH
                dst = out_hbm.at[pl.ds(t * B + b0, CH)]

                @pl.when(un < hi)
                def _():
                    pltpu.make_async_copy(rows_v.at[s], dst, osem[s]).wait()
                    start_gather(un, s)
            return carry

        lax.fori_loop(0, per_w // NBUF, pair, 0, unroll=False)

        for s in range(NBUF):
            u = hi - NBUF + s
            t = u // n_chunks_b
            b0 = (u % n_chunks_b) * CH
            dst = out_hbm.at[pl.ds(t * B + b0, CH)]
            pltpu.make_async_copy(rows_v.at[s], dst, osem[s]).wait()
        pltpu.sync_copy(acc_v, part_hbm.at[wid])

    return k(tab_p, idxT, tgtT, lse)


SPLITS = ((0, 25), (25, 25))


def kernel(idx, targets, table):
    Bq, Tq = idx.shape
    n = Bq * Tq
    idxT = idx.astype(jnp.int32).T
    tgtT = targets.astype(jnp.int32).T
    table = table.astype(jnp.float32)
    tab_p = jnp.pad(table, ((0, 0), (0, CP - C)))
    lse = _row_lse(table).reshape(V)
    halves = [_sc_gather(tab_p, idxT, tgtT, lse, ta, ts)
              for ta, ts in SPLITS]
    out3 = None
    for (ta, ts), (o, _) in zip(SPLITS, halves):
        o8 = o.reshape(ts * Bq * 8, 128)
        out3 = _tc_transpose_part(o8, out3, ta, ts, Bq, Tq)
    logits = jnp.transpose(out3, (2, 0, 1))
    loss = _finalize_loss([p for _, p in halves], n)
    return logits, loss
